# Initial kernel scaffold; baseline (speedup 1.0000x reference)
import jax
import jax.numpy as jnp
from jax.experimental import pallas as pl


def kernel(x, edge_index, batch, eps, W1, b1, g1, be1, W2, b2, go, bo):
    raise NotImplementedError("write your pallas kernel here")


# speedup vs baseline: 7.5132x; 7.5132x over previous
import functools

import jax
import jax.numpy as jnp
from jax import lax
from jax.experimental import pallas as pl
from jax.experimental.pallas import tpu as pltpu
from jax.experimental.pallas import tpu_sc as plsc

N = 10000
E = 320000
D = 128
H = 128
L = 3
G = 64

NC = 2
NS = 16
NW = NC * NS
EPW = E // NW
BE = 125
BLOCKS = EPW // BE
ROWS_PER_TILE = N // NS


WB = 624


def _seg_sum_body(h_hbm, src_hbm, dst_hbm, out_hbm,
                  src_v, dst_v, rows_v, zbuf_v, aggr_sh, sem):
    c = lax.axis_index("c")
    s = lax.axis_index("s")
    wid = c * NS + s

    @pl.loop(0, 16)
    def _(i):
        @pl.loop(0, D // 16)
        def _(j):
            zbuf_v[i, pl.ds(j * 16, 16)] = jnp.zeros((16,), jnp.float32)

    @pl.loop(0, WB // 16)
    def _(k):
        pltpu.sync_copy(zbuf_v, aggr_sh.at[pl.ds(s * WB + k * 16, 16)])

    @pl.when(s == NS - 1)
    def _():
        pltpu.sync_copy(zbuf_v, aggr_sh.at[pl.ds(N - 16, 16)])

    pltpu.async_copy(src_hbm.at[wid], src_v, sem).wait()
    pltpu.async_copy(dst_hbm.at[wid], dst_v, sem).wait()

    plsc.subcore_barrier()

    @pl.loop(0, BLOCKS)
    def _(j):
        pltpu.sync_copy(h_hbm.at[src_v.at[j]], rows_v)
        pltpu.sync_copy(rows_v, aggr_sh.at[dst_v.at[j]], add=True)

    plsc.subcore_barrier()

    @pl.when(s < NS - 1)
    def _():
        pltpu.sync_copy(aggr_sh.at[pl.ds(s * WB, WB)],
                        out_hbm.at[c, pl.ds(s * WB, WB)])

    @pl.when(s == NS - 1)
    def _():
        pltpu.sync_copy(aggr_sh.at[pl.ds((NS - 1) * WB, N - (NS - 1) * WB)],
                        out_hbm.at[c, pl.ds((NS - 1) * WB, N - (NS - 1) * WB)])


@jax.jit
def _sc_segment_sum(h, src, dst):
    mesh = plsc.VectorSubcoreMesh(core_axis_name="c", subcore_axis_name="s")
    k = pl.kernel(
        _seg_sum_body,
        out_type=jax.ShapeDtypeStruct((NC, N, D), jnp.float32),
        mesh=mesh,
        scratch_types=[
            pltpu.VMEM((BLOCKS, BE), jnp.int32),
            pltpu.VMEM((BLOCKS, BE), jnp.int32),
            pltpu.VMEM((BE, D), jnp.float32),
            pltpu.VMEM((16, D), jnp.float32),
            pltpu.VMEM_SHARED((N, D), jnp.float32),
            pltpu.SemaphoreType.DMA,
        ],
    )
    return k(h, src, dst)


def _mlp_body(h_ref, a_ref, eps_ref, w1_ref, b1_ref, g1_ref, be1_ref,
              w2_ref, b2_ref, go_ref, bo_ref, o_ref):
    e = eps_ref[0, 0]
    z = (1.0 + e) * h_ref[...] + a_ref[0] + a_ref[1]
    t = jnp.dot(z, w1_ref[...], preferred_element_type=jnp.float32)
    t = t + b1_ref[...]
    mu = jnp.mean(t, axis=0, keepdims=True)
    var = jnp.mean((t - mu) ** 2, axis=0, keepdims=True)
    t = g1_ref[...] * (t - mu) * lax.rsqrt(var + 1e-5) + be1_ref[...]
    t = jnp.maximum(t, 0.0)
    u = jnp.dot(t, w2_ref[...], preferred_element_type=jnp.float32)
    u = u + b2_ref[...]
    mu2 = jnp.mean(u, axis=0, keepdims=True)
    var2 = jnp.mean((u - mu2) ** 2, axis=0, keepdims=True)
    u = go_ref[...] * (u - mu2) * lax.rsqrt(var2 + 1e-5) + bo_ref[...]
    o_ref[...] = jnp.maximum(u, 0.0)


@jax.jit
def _tc_mlp(h, aggr, eps_i, w1, b1, g1, be1, w2, b2, go, bo):
    return pl.pallas_call(
        _mlp_body,
        out_shape=jax.ShapeDtypeStruct((N, H), jnp.float32),
    )(h, aggr, eps_i, w1, b1, g1, be1, w2, b2, go, bo)


def _pool_body(rep_ref, batch_ref, o_ref):
    gid = lax.broadcasted_iota(jnp.int32, (G, N), 0)
    onehot = jnp.where(gid == batch_ref[...], 1.0, 0.0)
    o_ref[...] = jnp.dot(onehot, rep_ref[...],
                         preferred_element_type=jnp.float32)


@jax.jit
def _tc_pool(rep, batch2d):
    return pl.pallas_call(
        _pool_body,
        out_shape=jax.ShapeDtypeStruct((G, rep.shape[1]), jnp.float32),
    )(rep, batch2d)


def kernel(x, edge_index, batch, eps, W1, b1, g1, be1, W2, b2, go, bo):
    src = edge_index[0].reshape(NW, BLOCKS, BE)
    dst = edge_index[1].reshape(NW, BLOCKS, BE)
    batch2d = batch.reshape(1, N)
    h = x
    reps = []
    for i in range(L):
        aggr = _sc_segment_sum(h, src, dst)
        h = _tc_mlp(h, aggr, eps[i].reshape(1, 1), W1[i],
                    b1[i].reshape(1, H), g1[i].reshape(1, H),
                    be1[i].reshape(1, H), W2[i], b2[i].reshape(1, H),
                    go[i].reshape(1, H), bo[i].reshape(1, H))
        reps.append(h)
    node_rep = jnp.concatenate(reps, axis=1)
    graph_rep = _tc_pool(node_rep, batch2d)
    return (graph_rep, node_rep)
